# baseline (speedup 1.0000x reference)
import functools

import jax
import jax.numpy as jnp
from jax import lax
from jax.experimental import pallas as pl
from jax.experimental.pallas import tpu as pltpu
from jax.experimental.pallas import tpu_sc as plsc

N = 10000
F = 128
FH = F // 2
E = 320000
CHUNK = 128
N_TILES = 16
N_CHUNKS = 2560
E_PAD = N_CHUNKS * CHUNK
SCH = 256
N_OPS = E_PAD // SCH
OPT = N_OPS // N_TILES
N_GRP = 4
OPG = OPT // N_GRP
NBUF = 2
TRASH = N
N_ACC = 10240
N_H = 10240
H_STRIPE = N_H // N_TILES
O_STRIPE = N_ACC // N_TILES

_mesh = plsc.VectorSubcoreMesh(core_axis_name="c", subcore_axis_name="s")


@jax.jit
def _deg(idx4, ones_v):
    CPT = N_CHUNKS // N_TILES

    @functools.partial(
        pl.kernel,
        mesh=_mesh,
        out_type=jax.ShapeDtypeStruct((4, N_H), jnp.float32),
        scratch_types=[
            pltpu.VMEM((CPT, CHUNK), jnp.int32),
            pltpu.VMEM((CHUNK,), jnp.float32),
            pltpu.VMEM((H_STRIPE,), jnp.float32),
            pltpu.VMEM_SHARED((N_H,), jnp.float32),
            pltpu.VMEM_SHARED((N_H,), jnp.float32),
        ],
    )
    def k(idx_hbm, ones_hbm, deg_hbm, idx_v, ones_vm, zb, hist0, hist1):
        c = lax.axis_index("c")
        s = lax.axis_index("s")
        pltpu.sync_copy(ones_hbm, ones_vm)
        @pl.loop(0, H_STRIPE, step=16)
        def _(i):
            zb[pl.ds(i, 16)] = jnp.zeros((16,), jnp.float32)

        for kk, hist in ((0, hist0), (1, hist1)):
            pltpu.sync_copy(zb, hist.at[pl.ds(s * H_STRIPE, H_STRIPE)])
        plsc.subcore_barrier()
        for kk, hist in ((0, hist0), (1, hist1)):
            pltpu.sync_copy(idx_hbm.at[2 * c + kk].at[pl.ds(s * CPT, CPT)], idx_v)

            @pl.loop(0, CPT)
            def _(j):
                pltpu.sync_copy(ones_vm, hist.at[idx_v.at[j]], add=True)

        plsc.subcore_barrier()
        for kk, hist in ((0, hist0), (1, hist1)):
            pltpu.sync_copy(hist.at[pl.ds(s * H_STRIPE, H_STRIPE)],
                            deg_hbm.at[2 * c + kk].at[pl.ds(s * H_STRIPE, H_STRIPE)])

    return k(idx4, ones_v)


@jax.jit
def _spmv(tables, srcg, dstg):
    @functools.partial(
        pl.kernel,
        mesh=_mesh,
        out_type=jax.ShapeDtypeStruct((2, 2, N_ACC, FH), jnp.float32),
        compiler_params=pltpu.CompilerParams(use_tc_tiling_on_sc=False),
        scratch_types=[
            pltpu.VMEM((OPG, SCH), jnp.int32),
            pltpu.VMEM((OPG, SCH), jnp.int32),
            pltpu.VMEM((SCH, FH), jnp.float32),
            pltpu.VMEM((SCH, FH), jnp.float32),
            pltpu.VMEM((16, FH), jnp.float32),
            pltpu.VMEM_SHARED((N_ACC, FH), jnp.float32),
            pltpu.VMEM_SHARED((N, FH), jnp.float32),
            pltpu.SemaphoreType.DMA,
            pltpu.SemaphoreType.DMA,
            pltpu.SemaphoreType.DMA,
            pltpu.SemaphoreType.DMA,
        ],
    )
    def k(t_hbm, src_hbm, dst_hbm, out_hbm, src_v, dst_v,
          b0, b1, zb, acc, tbl,
          g0, g1, s0, s1):
        c = lax.axis_index("c")
        s = lax.axis_index("s")
        bufs = (b0, b1)
        gsem = (g0, g1)
        ssem = (s0, s1)

        @pl.loop(0, 16)
        def _(r):
            for col in range(FH // 16):
                zb[r, pl.ds(col * 16, 16)] = jnp.zeros((16,), jnp.float32)

        zbase = s * O_STRIPE

        for rel in range(2):
            out = out_hbm.at[rel].at[c]

            pltpu.sync_copy(t_hbm.at[rel].at[c].at[pl.ds(s * 625, 625)],
                            tbl.at[pl.ds(s * 625, 625)])

            @pl.loop(0, O_STRIPE, step=16)
            def _(z):
                pltpu.sync_copy(zb, acc.at[pl.ds(zbase + z, 16)])

            plsc.subcore_barrier()
            tabs = (t_hbm.at[rel].at[c], tbl)

            @pl.loop(0, N_GRP)
            def _(grp):
                obase = s * OPT + grp * OPG
                pltpu.sync_copy(src_hbm.at[rel].at[pl.ds(obase, OPG)], src_v)
                pltpu.sync_copy(dst_hbm.at[rel].at[pl.ds(obase, OPG)], dst_v)
                for b in range(NBUF):
                    pltpu.async_copy(tabs[b].at[src_v.at[b]], bufs[b], gsem[b])

                @pl.loop(0, OPG, step=NBUF)
                def _(mb):
                    for b in range(NBUF):
                        m = mb + b
                        pltpu.make_async_copy(tabs[b].at[src_v.at[m]],
                                              bufs[b], gsem[b]).wait()
                        pltpu.async_copy(bufs[b], acc.at[dst_v.at[m]],
                                         ssem[b], add=True)
                    for b in range(NBUF):
                        m4 = mb + b + NBUF
                        pltpu.make_async_copy(bufs[b],
                                              acc.at[dst_v.at[mb + b]],
                                              ssem[b]).wait()

                        @pl.when(m4 < OPG)
                        def _():
                            pltpu.async_copy(tabs[b].at[src_v.at[m4]],
                                             bufs[b], gsem[b])

            plsc.subcore_barrier()
            pltpu.sync_copy(acc.at[pl.ds(s * O_STRIPE, O_STRIPE)],
                            out.at[pl.ds(s * O_STRIPE, O_STRIPE)])
            plsc.subcore_barrier()

    return k(tables, srcg, dstg)


def _norms(deg_ref):
    nrm = lax.rsqrt(jnp.maximum(deg_ref[...], 1.0))
    return (nrm[:, 0:1], nrm[:, 1:2], nrm[:, 2:3], nrm[:, 3:4])


BR = 2000


def _halves(a_ref, r):
    return jnp.concatenate([a_ref[r, 0], a_ref[r, 1]], axis=1)


@jax.jit
def _tc_pre(x, deg_t):
    def body(x_ref, d_ref, o_ref):
        nss, _, nsu, _ = _norms(d_ref)
        xv = x_ref[...]
        xs_s = xv * nss
        xs_u = xv * nsu
        o_ref[0, 0] = xs_s[:, :FH]
        o_ref[0, 1] = xs_s[:, FH:]
        o_ref[1, 0] = xs_u[:, :FH]
        o_ref[1, 1] = xs_u[:, FH:]

    return pl.pallas_call(
        body,
        grid=(N // BR,),
        in_specs=[pl.BlockSpec((BR, F), lambda i: (i, 0)),
                  pl.BlockSpec((BR, 4), lambda i: (i, 0))],
        out_specs=pl.BlockSpec((2, 2, BR, FH), lambda i: (0, 0, i, 0)),
        out_shape=jax.ShapeDtypeStruct((2, 2, N, FH), jnp.float32),
    )(x, deg_t)


@jax.jit
def _tc_mid(agg, deg_t, W1s, W1u, b1):
    def body(a_ref, d_ref, ws_ref, wu_ref, b_ref, o_ref):
        nss, nds, nsu, ndu = _norms(d_ref)
        h = jnp.dot(_halves(a_ref, 0) * nds, ws_ref[...],
                    preferred_element_type=jnp.float32)
        h += jnp.dot(_halves(a_ref, 1) * ndu, wu_ref[...],
                     preferred_element_type=jnp.float32)
        h = jnp.maximum(h + b_ref[...], 0.0)
        hs_s = h * nss
        hs_u = h * nsu
        o_ref[0, 0] = hs_s[:, :FH]
        o_ref[0, 1] = hs_s[:, FH:]
        o_ref[1, 0] = hs_u[:, :FH]
        o_ref[1, 1] = hs_u[:, FH:]

    return pl.pallas_call(
        body,
        grid=(N // BR,),
        in_specs=[pl.BlockSpec((2, 2, BR, FH), lambda i: (0, 0, i, 0)),
                  pl.BlockSpec((BR, 4), lambda i: (i, 0)),
                  pl.BlockSpec((F, F), lambda i: (0, 0)),
                  pl.BlockSpec((F, F), lambda i: (0, 0)),
                  pl.BlockSpec((1, F), lambda i: (0, 0))],
        out_specs=pl.BlockSpec((2, 2, BR, FH), lambda i: (0, 0, i, 0)),
        out_shape=jax.ShapeDtypeStruct((2, 2, N, FH), jnp.float32),
    )(agg, deg_t, W1s, W1u, b1.reshape(1, F))


@jax.jit
def _tc_out(agg, deg_t, W2s, W2u, b2):
    def body(a_ref, d_ref, ws_ref, wu_ref, b_ref, o_ref):
        _, nds, _, ndu = _norms(d_ref)
        o = jnp.dot(_halves(a_ref, 0) * nds, ws_ref[...],
                    preferred_element_type=jnp.float32)
        o += jnp.dot(_halves(a_ref, 1) * ndu, wu_ref[...],
                     preferred_element_type=jnp.float32)
        o_ref[...] = o + b_ref[...]

    return pl.pallas_call(
        body,
        grid=(N // BR,),
        in_specs=[pl.BlockSpec((2, 2, BR, FH), lambda i: (0, 0, i, 0)),
                  pl.BlockSpec((BR, 4), lambda i: (i, 0)),
                  pl.BlockSpec((F, F), lambda i: (0, 0)),
                  pl.BlockSpec((F, F), lambda i: (0, 0)),
                  pl.BlockSpec((1, F), lambda i: (0, 0))],
        out_specs=pl.BlockSpec((BR, F), lambda i: (i, 0)),
        out_shape=jax.ShapeDtypeStruct((N, F), jnp.float32),
    )(agg, deg_t, W2s, W2u, b2.reshape(1, F))


def kernel(x, edge_index_side, edge_index_upd,
           W1_side, b1_side, W1_upd, b1_upd,
           W2_side, b2_side, W2_upd, b2_upd):
    src_s = edge_index_side[0].astype(jnp.int32)
    dst_s = edge_index_side[1].astype(jnp.int32)
    src_u = edge_index_upd[0].astype(jnp.int32)
    dst_u = edge_index_upd[1].astype(jnp.int32)

    def pad2(a, fill):
        return jnp.concatenate(
            [a, jnp.full((E_PAD - E,), fill, jnp.int32)]).reshape(N_CHUNKS, CHUNK)

    def pad4(a, fill):
        return jnp.concatenate(
            [a, jnp.full((E_PAD - E,), fill, jnp.int32)]).reshape(N_OPS, SCH)

    idx4 = jnp.stack([pad2(src_s, TRASH), pad2(dst_s, TRASH),
                      pad2(src_u, TRASH), pad2(dst_u, TRASH)])
    srcg = jnp.stack([pad4(src_s, 0), pad4(src_u, 0)])
    dstg = jnp.stack([pad4(dst_s, TRASH), pad4(dst_u, TRASH)])

    degs = _deg(idx4, jnp.ones((CHUNK,), jnp.float32))
    deg_t = degs.T[:N]

    xs = _tc_pre(x, deg_t)
    agg1 = _spmv(xs, srcg, dstg)[:, :, :N]
    hs = _tc_mid(agg1, deg_t, W1_side, W1_upd, b1_side + b1_upd)
    agg2 = _spmv(hs, srcg, dstg)[:, :, :N]
    return _tc_out(agg2, deg_t, W2_side, W2_upd, b2_side + b2_upd)


# speedup vs baseline: 1.1763x; 1.1763x over previous
import functools

import jax
import jax.numpy as jnp
from jax import lax
from jax.experimental import pallas as pl
from jax.experimental.pallas import tpu as pltpu
from jax.experimental.pallas import tpu_sc as plsc

N = 10000
F = 128
FH = F // 2
E = 320000
CHUNK = 128
N_TILES = 16
N_CHUNKS = 2560
E_PAD = N_CHUNKS * CHUNK
SCH = 128
N_OPS = E_PAD // SCH
OPT = N_OPS // N_TILES
N_GRP = 4
OPG = OPT // N_GRP
NBUF = 4
TRASH = N
N_ACC = 10240
N_H = 10240
H_STRIPE = N_H // N_TILES
O_STRIPE = N_ACC // N_TILES

_mesh = plsc.VectorSubcoreMesh(core_axis_name="c", subcore_axis_name="s")


@jax.jit
def _deg(idx4, ones_v):
    CPT = N_CHUNKS // N_TILES

    @functools.partial(
        pl.kernel,
        mesh=_mesh,
        out_type=jax.ShapeDtypeStruct((4, N_H), jnp.float32),
        scratch_types=[
            pltpu.VMEM((CPT, CHUNK), jnp.int32),
            pltpu.VMEM((CHUNK,), jnp.float32),
            pltpu.VMEM((H_STRIPE,), jnp.float32),
            pltpu.VMEM_SHARED((N_H,), jnp.float32),
            pltpu.VMEM_SHARED((N_H,), jnp.float32),
        ],
    )
    def k(idx_hbm, ones_hbm, deg_hbm, idx_v, ones_vm, zb, hist0, hist1):
        c = lax.axis_index("c")
        s = lax.axis_index("s")
        pltpu.sync_copy(ones_hbm, ones_vm)
        @pl.loop(0, H_STRIPE, step=16)
        def _(i):
            zb[pl.ds(i, 16)] = jnp.zeros((16,), jnp.float32)

        for kk, hist in ((0, hist0), (1, hist1)):
            pltpu.sync_copy(zb, hist.at[pl.ds(s * H_STRIPE, H_STRIPE)])
        plsc.subcore_barrier()
        for kk, hist in ((0, hist0), (1, hist1)):
            pltpu.sync_copy(idx_hbm.at[2 * c + kk].at[pl.ds(s * CPT, CPT)], idx_v)

            @pl.loop(0, CPT)
            def _(j):
                pltpu.sync_copy(ones_vm, hist.at[idx_v.at[j]], add=True)

        plsc.subcore_barrier()
        for kk, hist in ((0, hist0), (1, hist1)):
            pltpu.sync_copy(hist.at[pl.ds(s * H_STRIPE, H_STRIPE)],
                            deg_hbm.at[2 * c + kk].at[pl.ds(s * H_STRIPE, H_STRIPE)])

    return k(idx4, ones_v)


@jax.jit
def _spmv(tables, srcg, dstg):
    @functools.partial(
        pl.kernel,
        mesh=_mesh,
        out_type=jax.ShapeDtypeStruct((2, 2, N_ACC, FH), jnp.float32),
        compiler_params=pltpu.CompilerParams(use_tc_tiling_on_sc=False),
        scratch_types=[
            pltpu.VMEM((OPG, SCH), jnp.int32),
            pltpu.VMEM((OPG, SCH), jnp.int32),
            pltpu.VMEM((SCH, FH), jnp.float32),
            pltpu.VMEM((SCH, FH), jnp.float32),
            pltpu.VMEM((SCH, FH), jnp.float32),
            pltpu.VMEM((SCH, FH), jnp.float32),
            pltpu.VMEM((16, FH), jnp.float32),
            pltpu.VMEM_SHARED((N_ACC, FH), jnp.float32),
            pltpu.VMEM_SHARED((N, FH), jnp.float32),
            pltpu.SemaphoreType.DMA,
            pltpu.SemaphoreType.DMA,
            pltpu.SemaphoreType.DMA,
            pltpu.SemaphoreType.DMA,
            pltpu.SemaphoreType.DMA,
            pltpu.SemaphoreType.DMA,
            pltpu.SemaphoreType.DMA,
            pltpu.SemaphoreType.DMA,
        ],
    )
    def k(t_hbm, src_hbm, dst_hbm, out_hbm, src_v, dst_v,
          b0, b1, b2, b3, zb, acc, tbl,
          g0, g1, g2, g3, s0, s1, s2, s3):
        c = lax.axis_index("c")
        s = lax.axis_index("s")
        bufs = (b0, b1, b2, b3)
        gsem = (g0, g1, g2, g3)
        ssem = (s0, s1, s2, s3)

        @pl.loop(0, 16)
        def _(r):
            for col in range(FH // 16):
                zb[r, pl.ds(col * 16, 16)] = jnp.zeros((16,), jnp.float32)

        zbase = s * O_STRIPE

        for rel in range(2):
            out = out_hbm.at[rel].at[c]

            pltpu.sync_copy(t_hbm.at[rel].at[c].at[pl.ds(s * 625, 625)],
                            tbl.at[pl.ds(s * 625, 625)])

            @pl.loop(0, O_STRIPE, step=16)
            def _(z):
                pltpu.sync_copy(zb, acc.at[pl.ds(zbase + z, 16)])

            plsc.subcore_barrier()
            th = t_hbm.at[rel].at[c]
            tabs = (th, tbl, th, tbl)

            @pl.loop(0, N_GRP)
            def _(grp):
                obase = s * OPT + grp * OPG
                pltpu.sync_copy(src_hbm.at[rel].at[pl.ds(obase, OPG)], src_v)
                pltpu.sync_copy(dst_hbm.at[rel].at[pl.ds(obase, OPG)], dst_v)
                for b in range(NBUF):
                    pltpu.async_copy(tabs[b].at[src_v.at[b]], bufs[b], gsem[b])

                @pl.loop(0, OPG, step=NBUF)
                def _(mb):
                    for b in range(NBUF):
                        m = mb + b
                        pltpu.make_async_copy(tabs[b].at[src_v.at[m]],
                                              bufs[b], gsem[b]).wait()
                        pltpu.async_copy(bufs[b], acc.at[dst_v.at[m]],
                                         ssem[b], add=True)
                    for b in range(NBUF):
                        m4 = mb + b + NBUF
                        pltpu.make_async_copy(bufs[b],
                                              acc.at[dst_v.at[mb + b]],
                                              ssem[b]).wait()

                        @pl.when(m4 < OPG)
                        def _():
                            pltpu.async_copy(tabs[b].at[src_v.at[m4]],
                                             bufs[b], gsem[b])

            plsc.subcore_barrier()
            pltpu.sync_copy(acc.at[pl.ds(s * O_STRIPE, O_STRIPE)],
                            out.at[pl.ds(s * O_STRIPE, O_STRIPE)])
            plsc.subcore_barrier()

    return k(tables, srcg, dstg)


def _norms(deg_ref):
    nrm = lax.rsqrt(jnp.maximum(deg_ref[...], 1.0))
    return (nrm[:, 0:1], nrm[:, 1:2], nrm[:, 2:3], nrm[:, 3:4])


BR = 2000


def _halves(a_ref, r):
    return jnp.concatenate([a_ref[r, 0], a_ref[r, 1]], axis=1)


@jax.jit
def _tc_pre(x, deg_t):
    def body(x_ref, d_ref, o_ref):
        nss, _, nsu, _ = _norms(d_ref)
        xv = x_ref[...]
        xs_s = xv * nss
        xs_u = xv * nsu
        o_ref[0, 0] = xs_s[:, :FH]
        o_ref[0, 1] = xs_s[:, FH:]
        o_ref[1, 0] = xs_u[:, :FH]
        o_ref[1, 1] = xs_u[:, FH:]

    return pl.pallas_call(
        body,
        grid=(N // BR,),
        in_specs=[pl.BlockSpec((BR, F), lambda i: (i, 0)),
                  pl.BlockSpec((BR, 4), lambda i: (i, 0))],
        out_specs=pl.BlockSpec((2, 2, BR, FH), lambda i: (0, 0, i, 0)),
        out_shape=jax.ShapeDtypeStruct((2, 2, N, FH), jnp.float32),
    )(x, deg_t)


@jax.jit
def _tc_mid(agg, deg_t, W1s, W1u, b1):
    def body(a_ref, d_ref, ws_ref, wu_ref, b_ref, o_ref):
        nss, nds, nsu, ndu = _norms(d_ref)
        h = jnp.dot(_halves(a_ref, 0) * nds, ws_ref[...],
                    preferred_element_type=jnp.float32)
        h += jnp.dot(_halves(a_ref, 1) * ndu, wu_ref[...],
                     preferred_element_type=jnp.float32)
        h = jnp.maximum(h + b_ref[...], 0.0)
        hs_s = h * nss
        hs_u = h * nsu
        o_ref[0, 0] = hs_s[:, :FH]
        o_ref[0, 1] = hs_s[:, FH:]
        o_ref[1, 0] = hs_u[:, :FH]
        o_ref[1, 1] = hs_u[:, FH:]

    return pl.pallas_call(
        body,
        grid=(N // BR,),
        in_specs=[pl.BlockSpec((2, 2, BR, FH), lambda i: (0, 0, i, 0)),
                  pl.BlockSpec((BR, 4), lambda i: (i, 0)),
                  pl.BlockSpec((F, F), lambda i: (0, 0)),
                  pl.BlockSpec((F, F), lambda i: (0, 0)),
                  pl.BlockSpec((1, F), lambda i: (0, 0))],
        out_specs=pl.BlockSpec((2, 2, BR, FH), lambda i: (0, 0, i, 0)),
        out_shape=jax.ShapeDtypeStruct((2, 2, N, FH), jnp.float32),
    )(agg, deg_t, W1s, W1u, b1.reshape(1, F))


@jax.jit
def _tc_out(agg, deg_t, W2s, W2u, b2):
    def body(a_ref, d_ref, ws_ref, wu_ref, b_ref, o_ref):
        _, nds, _, ndu = _norms(d_ref)
        o = jnp.dot(_halves(a_ref, 0) * nds, ws_ref[...],
                    preferred_element_type=jnp.float32)
        o += jnp.dot(_halves(a_ref, 1) * ndu, wu_ref[...],
                     preferred_element_type=jnp.float32)
        o_ref[...] = o + b_ref[...]

    return pl.pallas_call(
        body,
        grid=(N // BR,),
        in_specs=[pl.BlockSpec((2, 2, BR, FH), lambda i: (0, 0, i, 0)),
                  pl.BlockSpec((BR, 4), lambda i: (i, 0)),
                  pl.BlockSpec((F, F), lambda i: (0, 0)),
                  pl.BlockSpec((F, F), lambda i: (0, 0)),
                  pl.BlockSpec((1, F), lambda i: (0, 0))],
        out_specs=pl.BlockSpec((BR, F), lambda i: (i, 0)),
        out_shape=jax.ShapeDtypeStruct((N, F), jnp.float32),
    )(agg, deg_t, W2s, W2u, b2.reshape(1, F))


def kernel(x, edge_index_side, edge_index_upd,
           W1_side, b1_side, W1_upd, b1_upd,
           W2_side, b2_side, W2_upd, b2_upd):
    src_s = edge_index_side[0].astype(jnp.int32)
    dst_s = edge_index_side[1].astype(jnp.int32)
    src_u = edge_index_upd[0].astype(jnp.int32)
    dst_u = edge_index_upd[1].astype(jnp.int32)

    def pad2(a, fill):
        return jnp.concatenate(
            [a, jnp.full((E_PAD - E,), fill, jnp.int32)]).reshape(N_CHUNKS, CHUNK)

    def pad4(a, fill):
        return jnp.concatenate(
            [a, jnp.full((E_PAD - E,), fill, jnp.int32)]).reshape(N_OPS, SCH)

    idx4 = jnp.stack([pad2(src_s, TRASH), pad2(dst_s, TRASH),
                      pad2(src_u, TRASH), pad2(dst_u, TRASH)])
    srcg = jnp.stack([pad4(src_s, 0), pad4(src_u, 0)])
    dstg = jnp.stack([pad4(dst_s, TRASH), pad4(dst_u, TRASH)])

    degs = _deg(idx4, jnp.ones((CHUNK,), jnp.float32))
    deg_t = degs.T[:N]

    xs = _tc_pre(x, deg_t)
    agg1 = _spmv(xs, srcg, dstg)[:, :, :N]
    hs = _tc_mid(agg1, deg_t, W1_side, W1_upd, b1_side + b1_upd)
    agg2 = _spmv(hs, srcg, dstg)[:, :, :N]
    return _tc_out(agg2, deg_t, W2_side, W2_upd, b2_side + b2_upd)


# speedup vs baseline: 1.3425x; 1.1413x over previous
import functools

import jax
import jax.numpy as jnp
from jax import lax
from jax.experimental import pallas as pl
from jax.experimental.pallas import tpu as pltpu
from jax.experimental.pallas import tpu_sc as plsc

N = 10000
F = 128
FH = F // 2
E = 320000
CHUNK = 128
DCH = 256
N_TILES = 16
N_CHUNKS = 2560
E_PAD = N_CHUNKS * CHUNK
SCH = 256
N_OPS = E_PAD // SCH
OPT = N_OPS // N_TILES
N_GRP = 4
OPG = OPT // N_GRP
NBUF = 2
TRASH = N
N_ACC = 10240
N_H = 10240
H_STRIPE = N_H // N_TILES
O_STRIPE = N_ACC // N_TILES

_mesh = plsc.VectorSubcoreMesh(core_axis_name="c", subcore_axis_name="s")


@jax.jit
def _deg(idx4, ones_v):
    CPT = E_PAD // DCH // N_TILES

    @functools.partial(
        pl.kernel,
        mesh=_mesh,
        out_type=jax.ShapeDtypeStruct((4, N_H), jnp.float32),
        compiler_params=pltpu.CompilerParams(use_tc_tiling_on_sc=False),
        scratch_types=[
            pltpu.VMEM((CPT, DCH), jnp.int32),
            pltpu.VMEM((DCH,), jnp.float32),
            pltpu.VMEM((H_STRIPE,), jnp.float32),
            pltpu.VMEM_SHARED((N_H,), jnp.float32),
            pltpu.VMEM_SHARED((N_H,), jnp.float32),
            pltpu.SemaphoreType.DMA,
        ],
    )
    def k(idx_hbm, ones_hbm, deg_hbm, idx_v, ones_vm, zb, hist0, hist1, dsem):
        c = lax.axis_index("c")
        s = lax.axis_index("s")
        pltpu.sync_copy(ones_hbm, ones_vm)
        @pl.loop(0, H_STRIPE, step=16)
        def _(i):
            zb[pl.ds(i, 16)] = jnp.zeros((16,), jnp.float32)

        for kk, hist in ((0, hist0), (1, hist1)):
            pltpu.sync_copy(zb, hist.at[pl.ds(s * H_STRIPE, H_STRIPE)])
        plsc.subcore_barrier()
        for kk, hist in ((0, hist0), (1, hist1)):
            pltpu.sync_copy(idx_hbm.at[2 * c + kk].at[pl.ds(s * CPT, CPT)], idx_v)

            @pl.loop(0, CPT, step=8)
            def _(j):
                for b in range(8):
                    pltpu.async_copy(ones_vm, hist.at[idx_v.at[j + b]],
                                     dsem, add=True)
                for b in range(8):
                    pltpu.make_async_copy(ones_vm, hist.at[idx_v.at[j]],
                                          dsem).wait()

        plsc.subcore_barrier()
        for kk, hist in ((0, hist0), (1, hist1)):
            pltpu.sync_copy(hist.at[pl.ds(s * H_STRIPE, H_STRIPE)],
                            deg_hbm.at[2 * c + kk].at[pl.ds(s * H_STRIPE, H_STRIPE)])

    return k(idx4, ones_v)


@jax.jit
def _spmv(tables, srcg, dstg):
    @functools.partial(
        pl.kernel,
        mesh=_mesh,
        out_type=jax.ShapeDtypeStruct((2, 2, N_ACC, FH), jnp.float32),
        compiler_params=pltpu.CompilerParams(use_tc_tiling_on_sc=False),
        scratch_types=[
            pltpu.VMEM((OPG, SCH), jnp.int32),
            pltpu.VMEM((OPG, SCH), jnp.int32),
            pltpu.VMEM((SCH, FH), jnp.float32),
            pltpu.VMEM((SCH, FH), jnp.float32),
            pltpu.VMEM((16, FH), jnp.float32),
            pltpu.VMEM_SHARED((N_ACC, FH), jnp.float32),
            pltpu.VMEM_SHARED((N, FH), jnp.float32),
            pltpu.SemaphoreType.DMA,
            pltpu.SemaphoreType.DMA,
            pltpu.SemaphoreType.DMA,
            pltpu.SemaphoreType.DMA,
        ],
    )
    def k(t_hbm, src_hbm, dst_hbm, out_hbm, src_v, dst_v,
          b0, b1, zb, acc, tbl,
          g0, g1, s0, s1):
        c = lax.axis_index("c")
        s = lax.axis_index("s")
        bufs = (b0, b1)
        gsem = (g0, g1)
        ssem = (s0, s1)

        @pl.loop(0, 16)
        def _(r):
            for col in range(FH // 16):
                zb[r, pl.ds(col * 16, 16)] = jnp.zeros((16,), jnp.float32)

        zbase = s * O_STRIPE

        for rel in range(2):
            out = out_hbm.at[rel].at[c]

            pltpu.sync_copy(t_hbm.at[rel].at[c].at[pl.ds(s * 625, 625)],
                            tbl.at[pl.ds(s * 625, 625)])

            @pl.loop(0, O_STRIPE, step=16)
            def _(z):
                pltpu.sync_copy(zb, acc.at[pl.ds(zbase + z, 16)])

            plsc.subcore_barrier()
            table = tbl

            @pl.loop(0, N_GRP)
            def _(grp):
                obase = s * OPT + grp * OPG
                pltpu.sync_copy(src_hbm.at[rel].at[pl.ds(obase, OPG)], src_v)
                pltpu.sync_copy(dst_hbm.at[rel].at[pl.ds(obase, OPG)], dst_v)
                for b in range(NBUF):
                    pltpu.async_copy(table.at[src_v.at[b]], bufs[b], gsem[b])

                @pl.loop(0, OPG, step=NBUF)
                def _(mb):
                    for b in range(NBUF):
                        m = mb + b
                        pltpu.make_async_copy(table.at[src_v.at[m]],
                                              bufs[b], gsem[b]).wait()
                        pltpu.async_copy(bufs[b], acc.at[dst_v.at[m]],
                                         ssem[b], add=True)
                    for b in range(NBUF):
                        m4 = mb + b + NBUF
                        pltpu.make_async_copy(bufs[b],
                                              acc.at[dst_v.at[mb + b]],
                                              ssem[b]).wait()

                        @pl.when(m4 < OPG)
                        def _():
                            pltpu.async_copy(table.at[src_v.at[m4]],
                                             bufs[b], gsem[b])

            plsc.subcore_barrier()
            pltpu.sync_copy(acc.at[pl.ds(s * O_STRIPE, O_STRIPE)],
                            out.at[pl.ds(s * O_STRIPE, O_STRIPE)])
            plsc.subcore_barrier()

    return k(tables, srcg, dstg)


def _norms(deg_ref):
    nrm = lax.rsqrt(jnp.maximum(deg_ref[...], 1.0))
    return (nrm[:, 0:1], nrm[:, 1:2], nrm[:, 2:3], nrm[:, 3:4])


BR = 2000


def _halves(a_ref, r):
    return jnp.concatenate([a_ref[r, 0], a_ref[r, 1]], axis=1)


@jax.jit
def _tc_pre(x, deg_t):
    def body(x_ref, d_ref, o_ref):
        nss, _, nsu, _ = _norms(d_ref)
        xv = x_ref[...]
        xs_s = xv * nss
        xs_u = xv * nsu
        o_ref[0, 0] = xs_s[:, :FH]
        o_ref[0, 1] = xs_s[:, FH:]
        o_ref[1, 0] = xs_u[:, :FH]
        o_ref[1, 1] = xs_u[:, FH:]

    return pl.pallas_call(
        body,
        grid=(N // BR,),
        in_specs=[pl.BlockSpec((BR, F), lambda i: (i, 0)),
                  pl.BlockSpec((BR, 4), lambda i: (i, 0))],
        out_specs=pl.BlockSpec((2, 2, BR, FH), lambda i: (0, 0, i, 0)),
        out_shape=jax.ShapeDtypeStruct((2, 2, N, FH), jnp.float32),
    )(x, deg_t)


@jax.jit
def _tc_mid(agg, deg_t, W1s, W1u, b1):
    def body(a_ref, d_ref, ws_ref, wu_ref, b_ref, o_ref):
        nss, nds, nsu, ndu = _norms(d_ref)
        h = jnp.dot(_halves(a_ref, 0) * nds, ws_ref[...],
                    preferred_element_type=jnp.float32)
        h += jnp.dot(_halves(a_ref, 1) * ndu, wu_ref[...],
                     preferred_element_type=jnp.float32)
        h = jnp.maximum(h + b_ref[...], 0.0)
        hs_s = h * nss
        hs_u = h * nsu
        o_ref[0, 0] = hs_s[:, :FH]
        o_ref[0, 1] = hs_s[:, FH:]
        o_ref[1, 0] = hs_u[:, :FH]
        o_ref[1, 1] = hs_u[:, FH:]

    return pl.pallas_call(
        body,
        grid=(N // BR,),
        in_specs=[pl.BlockSpec((2, 2, BR, FH), lambda i: (0, 0, i, 0)),
                  pl.BlockSpec((BR, 4), lambda i: (i, 0)),
                  pl.BlockSpec((F, F), lambda i: (0, 0)),
                  pl.BlockSpec((F, F), lambda i: (0, 0)),
                  pl.BlockSpec((1, F), lambda i: (0, 0))],
        out_specs=pl.BlockSpec((2, 2, BR, FH), lambda i: (0, 0, i, 0)),
        out_shape=jax.ShapeDtypeStruct((2, 2, N, FH), jnp.float32),
    )(agg, deg_t, W1s, W1u, b1.reshape(1, F))


@jax.jit
def _tc_out(agg, deg_t, W2s, W2u, b2):
    def body(a_ref, d_ref, ws_ref, wu_ref, b_ref, o_ref):
        _, nds, _, ndu = _norms(d_ref)
        o = jnp.dot(_halves(a_ref, 0) * nds, ws_ref[...],
                    preferred_element_type=jnp.float32)
        o += jnp.dot(_halves(a_ref, 1) * ndu, wu_ref[...],
                     preferred_element_type=jnp.float32)
        o_ref[...] = o + b_ref[...]

    return pl.pallas_call(
        body,
        grid=(N // BR,),
        in_specs=[pl.BlockSpec((2, 2, BR, FH), lambda i: (0, 0, i, 0)),
                  pl.BlockSpec((BR, 4), lambda i: (i, 0)),
                  pl.BlockSpec((F, F), lambda i: (0, 0)),
                  pl.BlockSpec((F, F), lambda i: (0, 0)),
                  pl.BlockSpec((1, F), lambda i: (0, 0))],
        out_specs=pl.BlockSpec((BR, F), lambda i: (i, 0)),
        out_shape=jax.ShapeDtypeStruct((N, F), jnp.float32),
    )(agg, deg_t, W2s, W2u, b2.reshape(1, F))


def kernel(x, edge_index_side, edge_index_upd,
           W1_side, b1_side, W1_upd, b1_upd,
           W2_side, b2_side, W2_upd, b2_upd):
    src_s = edge_index_side[0].astype(jnp.int32)
    dst_s = edge_index_side[1].astype(jnp.int32)
    src_u = edge_index_upd[0].astype(jnp.int32)
    dst_u = edge_index_upd[1].astype(jnp.int32)

    def pad2(a, fill):
        return jnp.concatenate(
            [a, jnp.full((E_PAD - E,), fill, jnp.int32)]).reshape(N_CHUNKS, CHUNK)

    def pad4(a, fill):
        return jnp.concatenate(
            [a, jnp.full((E_PAD - E,), fill, jnp.int32)]).reshape(N_OPS, SCH)

    def padd(a, fill):
        return jnp.concatenate(
            [a, jnp.full((E_PAD - E,), fill, jnp.int32)]).reshape(E_PAD // DCH, DCH)

    idx4 = jnp.stack([padd(src_s, TRASH), padd(dst_s, TRASH),
                      padd(src_u, TRASH), padd(dst_u, TRASH)])
    srcg = jnp.stack([pad4(src_s, 0), pad4(src_u, 0)])
    dstg = jnp.stack([pad4(dst_s, TRASH), pad4(dst_u, TRASH)])

    degs = _deg(idx4, jnp.ones((DCH,), jnp.float32))
    deg_t = degs.T[:N]

    xs = _tc_pre(x, deg_t)
    agg1 = _spmv(xs, srcg, dstg)[:, :, :N]
    hs = _tc_mid(agg1, deg_t, W1_side, W1_upd, b1_side + b1_upd)
    agg2 = _spmv(hs, srcg, dstg)[:, :, :N]
    return _tc_out(agg2, deg_t, W2_side, W2_upd, b2_side + b2_upd)


# speedup vs baseline: 1.5167x; 1.1297x over previous
import functools

import jax
import jax.numpy as jnp
from jax import lax
from jax.experimental import pallas as pl
from jax.experimental.pallas import tpu as pltpu
from jax.experimental.pallas import tpu_sc as plsc

N = 10000
F = 128
FH = F // 2
E = 320000
CHUNK = 128
DCH = 256
N_TILES = 16
N_CHUNKS = 2560
E_PAD = N_CHUNKS * CHUNK
SCH = 320
N_OPS = E_PAD // SCH
OPT = N_OPS // N_TILES
N_GRP = 8
OPG = OPT // N_GRP
NBUF = 2
TRASH = N
N_ACC = 10240
N_H = 10240
H_STRIPE = N_H // N_TILES
O_STRIPE = N_ACC // N_TILES

_mesh = plsc.VectorSubcoreMesh(core_axis_name="c", subcore_axis_name="s")


@jax.jit
def _deg(idx4, ones_v):
    CPT = E_PAD // DCH // N_TILES

    @functools.partial(
        pl.kernel,
        mesh=_mesh,
        out_type=jax.ShapeDtypeStruct((4, N_H), jnp.float32),
        compiler_params=pltpu.CompilerParams(use_tc_tiling_on_sc=False),
        scratch_types=[
            pltpu.VMEM((CPT, DCH), jnp.int32),
            pltpu.VMEM((DCH,), jnp.float32),
            pltpu.VMEM((H_STRIPE,), jnp.float32),
            pltpu.VMEM_SHARED((N_H,), jnp.float32),
            pltpu.VMEM_SHARED((N_H,), jnp.float32),
            pltpu.SemaphoreType.DMA,
        ],
    )
    def k(idx_hbm, ones_hbm, deg_hbm, idx_v, ones_vm, zb, hist0, hist1, dsem):
        c = lax.axis_index("c")
        s = lax.axis_index("s")
        pltpu.sync_copy(ones_hbm, ones_vm)
        @pl.loop(0, H_STRIPE, step=16)
        def _(i):
            zb[pl.ds(i, 16)] = jnp.zeros((16,), jnp.float32)

        for kk, hist in ((0, hist0), (1, hist1)):
            pltpu.sync_copy(zb, hist.at[pl.ds(s * H_STRIPE, H_STRIPE)])
        plsc.subcore_barrier()
        for kk, hist in ((0, hist0), (1, hist1)):
            pltpu.sync_copy(idx_hbm.at[2 * c + kk].at[pl.ds(s * CPT, CPT)], idx_v)

            @pl.loop(0, CPT, step=8)
            def _(j):
                for b in range(8):
                    pltpu.async_copy(ones_vm, hist.at[idx_v.at[j + b]],
                                     dsem, add=True)
                for b in range(8):
                    pltpu.make_async_copy(ones_vm, hist.at[idx_v.at[j]],
                                          dsem).wait()

        plsc.subcore_barrier()
        for kk, hist in ((0, hist0), (1, hist1)):
            pltpu.sync_copy(hist.at[pl.ds(s * H_STRIPE, H_STRIPE)],
                            deg_hbm.at[2 * c + kk].at[pl.ds(s * H_STRIPE, H_STRIPE)])

    return k(idx4, ones_v)


@jax.jit
def _spmv(tables, srcg, dstg):
    @functools.partial(
        pl.kernel,
        mesh=_mesh,
        out_type=jax.ShapeDtypeStruct((2, 2, N_ACC, FH), jnp.float32),
        compiler_params=pltpu.CompilerParams(use_tc_tiling_on_sc=False),
        scratch_types=[
            pltpu.VMEM((OPG, SCH), jnp.int32),
            pltpu.VMEM((OPG, SCH), jnp.int32),
            pltpu.VMEM((SCH, FH), jnp.float32),
            pltpu.VMEM((SCH, FH), jnp.float32),
            pltpu.VMEM((16, FH), jnp.float32),
            pltpu.VMEM_SHARED((N_ACC, FH), jnp.float32),
            pltpu.VMEM_SHARED((N, FH), jnp.float32),
            pltpu.SemaphoreType.DMA,
            pltpu.SemaphoreType.DMA,
            pltpu.SemaphoreType.DMA,
            pltpu.SemaphoreType.DMA,
        ],
    )
    def k(t_hbm, src_hbm, dst_hbm, out_hbm, src_v, dst_v,
          b0, b1, zb, acc, tbl,
          g0, g1, s0, s1):
        c = lax.axis_index("c")
        s = lax.axis_index("s")
        bufs = (b0, b1)
        gsem = (g0, g1)
        ssem = (s0, s1)

        @pl.loop(0, 16)
        def _(r):
            for col in range(FH // 16):
                zb[r, pl.ds(col * 16, 16)] = jnp.zeros((16,), jnp.float32)

        zbase = s * O_STRIPE

        for rel in range(2):
            out = out_hbm.at[rel].at[c]

            pltpu.sync_copy(t_hbm.at[rel].at[c].at[pl.ds(s * 625, 625)],
                            tbl.at[pl.ds(s * 625, 625)])

            @pl.loop(0, O_STRIPE, step=16)
            def _(z):
                pltpu.sync_copy(zb, acc.at[pl.ds(zbase + z, 16)])

            plsc.subcore_barrier()
            table = tbl

            @pl.loop(0, N_GRP)
            def _(grp):
                obase = s * OPT + grp * OPG
                pltpu.sync_copy(src_hbm.at[rel].at[pl.ds(obase, OPG)], src_v)
                pltpu.sync_copy(dst_hbm.at[rel].at[pl.ds(obase, OPG)], dst_v)
                for b in range(NBUF):
                    pltpu.async_copy(table.at[src_v.at[b]], bufs[b], gsem[b])

                @pl.loop(0, OPG, step=NBUF)
                def _(mb):
                    for b in range(NBUF):
                        m = mb + b
                        pltpu.make_async_copy(table.at[src_v.at[m]],
                                              bufs[b], gsem[b]).wait()
                        pltpu.async_copy(bufs[b], acc.at[dst_v.at[m]],
                                         ssem[b], add=True)
                    for b in range(NBUF):
                        m4 = mb + b + NBUF
                        pltpu.make_async_copy(bufs[b],
                                              acc.at[dst_v.at[mb + b]],
                                              ssem[b]).wait()

                        @pl.when(m4 < OPG)
                        def _():
                            pltpu.async_copy(table.at[src_v.at[m4]],
                                             bufs[b], gsem[b])

            plsc.subcore_barrier()
            pltpu.sync_copy(acc.at[pl.ds(s * O_STRIPE, O_STRIPE)],
                            out.at[pl.ds(s * O_STRIPE, O_STRIPE)])
            plsc.subcore_barrier()

    return k(tables, srcg, dstg)


def _norms(deg_ref):
    nrm = lax.rsqrt(jnp.maximum(deg_ref[...], 1.0))
    return (nrm[:, 0:1], nrm[:, 1:2], nrm[:, 2:3], nrm[:, 3:4])


BR = 2000


def _halves(a_ref, r):
    return jnp.concatenate([a_ref[r, 0], a_ref[r, 1]], axis=1)


@jax.jit
def _tc_pre(x, deg_t):
    def body(x_ref, d_ref, o_ref):
        nss, _, nsu, _ = _norms(d_ref)
        xv = x_ref[...]
        xs_s = xv * nss
        xs_u = xv * nsu
        o_ref[0, 0] = xs_s[:, :FH]
        o_ref[0, 1] = xs_s[:, FH:]
        o_ref[1, 0] = xs_u[:, :FH]
        o_ref[1, 1] = xs_u[:, FH:]

    return pl.pallas_call(
        body,
        grid=(N // BR,),
        in_specs=[pl.BlockSpec((BR, F), lambda i: (i, 0)),
                  pl.BlockSpec((BR, 4), lambda i: (i, 0))],
        out_specs=pl.BlockSpec((2, 2, BR, FH), lambda i: (0, 0, i, 0)),
        out_shape=jax.ShapeDtypeStruct((2, 2, N, FH), jnp.float32),
    )(x, deg_t)


@jax.jit
def _tc_mid(agg, deg_t, W1s, W1u, b1):
    def body(a_ref, d_ref, ws_ref, wu_ref, b_ref, o_ref):
        nss, nds, nsu, ndu = _norms(d_ref)
        h = jnp.dot(_halves(a_ref, 0) * nds, ws_ref[...],
                    preferred_element_type=jnp.float32)
        h += jnp.dot(_halves(a_ref, 1) * ndu, wu_ref[...],
                     preferred_element_type=jnp.float32)
        h = jnp.maximum(h + b_ref[...], 0.0)
        hs_s = h * nss
        hs_u = h * nsu
        o_ref[0, 0] = hs_s[:, :FH]
        o_ref[0, 1] = hs_s[:, FH:]
        o_ref[1, 0] = hs_u[:, :FH]
        o_ref[1, 1] = hs_u[:, FH:]

    return pl.pallas_call(
        body,
        grid=(N // BR,),
        in_specs=[pl.BlockSpec((2, 2, BR, FH), lambda i: (0, 0, i, 0)),
                  pl.BlockSpec((BR, 4), lambda i: (i, 0)),
                  pl.BlockSpec((F, F), lambda i: (0, 0)),
                  pl.BlockSpec((F, F), lambda i: (0, 0)),
                  pl.BlockSpec((1, F), lambda i: (0, 0))],
        out_specs=pl.BlockSpec((2, 2, BR, FH), lambda i: (0, 0, i, 0)),
        out_shape=jax.ShapeDtypeStruct((2, 2, N, FH), jnp.float32),
    )(agg, deg_t, W1s, W1u, b1.reshape(1, F))


@jax.jit
def _tc_out(agg, deg_t, W2s, W2u, b2):
    def body(a_ref, d_ref, ws_ref, wu_ref, b_ref, o_ref):
        _, nds, _, ndu = _norms(d_ref)
        o = jnp.dot(_halves(a_ref, 0) * nds, ws_ref[...],
                    preferred_element_type=jnp.float32)
        o += jnp.dot(_halves(a_ref, 1) * ndu, wu_ref[...],
                     preferred_element_type=jnp.float32)
        o_ref[...] = o + b_ref[...]

    return pl.pallas_call(
        body,
        grid=(N // BR,),
        in_specs=[pl.BlockSpec((2, 2, BR, FH), lambda i: (0, 0, i, 0)),
                  pl.BlockSpec((BR, 4), lambda i: (i, 0)),
                  pl.BlockSpec((F, F), lambda i: (0, 0)),
                  pl.BlockSpec((F, F), lambda i: (0, 0)),
                  pl.BlockSpec((1, F), lambda i: (0, 0))],
        out_specs=pl.BlockSpec((BR, F), lambda i: (i, 0)),
        out_shape=jax.ShapeDtypeStruct((N, F), jnp.float32),
    )(agg, deg_t, W2s, W2u, b2.reshape(1, F))


def kernel(x, edge_index_side, edge_index_upd,
           W1_side, b1_side, W1_upd, b1_upd,
           W2_side, b2_side, W2_upd, b2_upd):
    src_s = edge_index_side[0].astype(jnp.int32)
    dst_s = edge_index_side[1].astype(jnp.int32)
    src_u = edge_index_upd[0].astype(jnp.int32)
    dst_u = edge_index_upd[1].astype(jnp.int32)

    def pad2(a, fill):
        return jnp.concatenate(
            [a, jnp.full((E_PAD - E,), fill, jnp.int32)]).reshape(N_CHUNKS, CHUNK)

    def pad4(a, fill):
        return jnp.concatenate(
            [a, jnp.full((E_PAD - E,), fill, jnp.int32)]).reshape(N_OPS, SCH)

    def padd(a, fill):
        return jnp.concatenate(
            [a, jnp.full((E_PAD - E,), fill, jnp.int32)]).reshape(E_PAD // DCH, DCH)

    idx4 = jnp.stack([padd(src_s, TRASH), padd(dst_s, TRASH),
                      padd(src_u, TRASH), padd(dst_u, TRASH)])
    srcg = jnp.stack([pad4(src_s, 0), pad4(src_u, 0)])
    dstg = jnp.stack([pad4(dst_s, TRASH), pad4(dst_u, TRASH)])

    degs = _deg(idx4, jnp.ones((DCH,), jnp.float32))
    deg_t = degs.T[:N]

    xs = _tc_pre(x, deg_t)
    agg1 = _spmv(xs, srcg, dstg)[:, :, :N]
    hs = _tc_mid(agg1, deg_t, W1_side, W1_upd, b1_side + b1_upd)
    agg2 = _spmv(hs, srcg, dstg)[:, :, :N]
    return _tc_out(agg2, deg_t, W2_side, W2_upd, b2_side + b2_upd)
